# baseline (speedup 1.0000x reference)
import functools

import jax
import jax.numpy as jnp
from jax import lax
from jax.experimental import pallas as pl
from jax.experimental.pallas import tpu as pltpu
from jax.experimental.pallas import tpu_sc as plsc

N = 10000
E = 320000
F = 128
NW = 32
CH = E // NW
K = 80
WPW = CH // K

_mesh = plsc.VectorSubcoreMesh(core_axis_name="c", subcore_axis_name="s")


@functools.partial(
    pl.kernel, mesh=_mesh,
    out_type=(jax.ShapeDtypeStruct((E,), jnp.float32),
              jax.ShapeDtypeStruct((2 * N,), jnp.float32)),
    scratch_types=[
        pltpu.VMEM((CH,), jnp.int32),
        pltpu.VMEM((CH,), jnp.int32),
        pltpu.VMEM((CH,), jnp.float32),
        pltpu.VMEM((CH,), jnp.float32),
        pltpu.VMEM((K,), jnp.int32),
        pltpu.VMEM((K,), jnp.int32),
        pltpu.VMEM((K,), jnp.float32),
        pltpu.VMEM((K,), jnp.float32),
        pltpu.VMEM((K,), jnp.float32),
        pltpu.VMEM((K,), jnp.float32),
        pltpu.VMEM((640,), jnp.float32),
        pltpu.VMEM_SHARED((N,), jnp.float32),
        pltpu.SemaphoreType.DMA,
        pltpu.SemaphoreType.DMA,
        pltpu.SemaphoreType.DMA,
        pltpu.SemaphoreType.DMA,
        pltpu.SemaphoreType.DMA,
        pltpu.SemaphoreType.DMA,
    ],
)
def _edge_w_deg(src_hbm, dst_hbm, ew_hbm, a1_hbm, a2_hbm, w_out, deg_out,
                sidx_all, didx_all, ewv, wv_all, didx0, didx1,
                a1v0, a1v1, a2v0, a2v1,
                zbuf, deg_sp, semA0, semA1, semB0, semB1, semD0, semD1):
    c = lax.axis_index("c")
    s = lax.axis_index("s")
    wid = s * 2 + c
    a1b = (a1v0, a1v1)
    a2b = (a2v0, a2v1)
    didxb = (didx0, didx1)
    semA = (semA0, semA1)
    semB = (semB0, semB1)
    semD = (semD0, semD1)

    chunk = pl.multiple_of(wid * CH, 8)
    pltpu.sync_copy(src_hbm.at[pl.ds(chunk, CH)], sidx_all)
    pltpu.sync_copy(dst_hbm.at[pl.ds(chunk, CH)], didx_all)
    pltpu.sync_copy(ew_hbm.at[pl.ds(chunk, CH)], ewv)
    pltpu.async_copy(a1_hbm.at[sidx_all.at[pl.ds(0, K)]], a1v0, semA0)
    pltpu.async_copy(a2_hbm.at[didx_all.at[pl.ds(0, K)]], a2v0, semB0)

    zero = jnp.zeros((16,), jnp.float32)
    for q in range(40):
        zbuf[pl.ds(q * 16, 16)] = zero
    base = pl.multiple_of(s * 624, 8)
    pltpu.sync_copy(zbuf.at[pl.ds(0, 624)], deg_sp.at[pl.ds(base, 624)])

    @pl.when(s == 0)
    def _():
        pltpu.sync_copy(zbuf.at[pl.ds(0, 16)], deg_sp.at[pl.ds(9984, 16)])

    plsc.subcore_barrier()

    def slot(t, rb, prefetch=True, warmup=False):
        off = pl.multiple_of(t * K, 16)
        pltpu.make_async_copy(a1_hbm.at[sidx_all.at[pl.ds(0, K)]],
                              a1b[rb], semA[rb]).wait()
        pltpu.make_async_copy(a2_hbm.at[didx_all.at[pl.ds(0, K)]],
                              a2b[rb], semB[rb]).wait()
        if prefetch:
            off1 = pl.multiple_of((t + 1) * K, 16)
            pltpu.async_copy(a1_hbm.at[sidx_all.at[pl.ds(off1, K)]],
                             a1b[1 - rb], semA[1 - rb])
            pltpu.async_copy(a2_hbm.at[didx_all.at[pl.ds(off1, K)]],
                             a2b[1 - rb], semB[1 - rb])
        for g in range(K // 16):
            sl = pl.ds(g * 16, 16)
            z = a1b[rb][sl] + a2b[rb][sl]
            wv_all[pl.ds(off + g * 16, 16)] = ewv[pl.ds(off + g * 16, 16)] \
                / (1.0 + jnp.exp(-z))
        if not warmup:
            pltpu.make_async_copy(wv_all.at[pl.ds(0, K)],
                                  deg_sp.at[didxb[rb]], semD[rb]).wait()
        for g in range(K // 16):
            didxb[rb][pl.ds(g * 16, 16)] = didx_all[pl.ds(off + g * 16, 16)]
        pltpu.async_copy(wv_all.at[pl.ds(off, K)], deg_sp.at[didxb[rb]],
                         semD[rb], add=True)

    slot(0, 0, warmup=True)
    slot(1, 1, warmup=True)

    def body(i, carry):
        t = i * 2 + 2
        slot(t, 0)
        slot(t + 1, 1)
        return carry

    lax.fori_loop(0, 61, body, 0)
    slot(WPW - 1, 0, prefetch=False)
    pltpu.make_async_copy(wv_all.at[pl.ds(0, K)], deg_sp.at[didxb[1]],
                          semD[1]).wait()
    pltpu.make_async_copy(wv_all.at[pl.ds(0, K)], deg_sp.at[didxb[0]],
                          semD[0]).wait()

    pltpu.sync_copy(wv_all, w_out.at[pl.ds(chunk, CH)])
    plsc.subcore_barrier()

    obase = pl.multiple_of(c * N + s * 624, 8)
    pltpu.sync_copy(deg_sp.at[pl.ds(base, 624)], zbuf.at[pl.ds(0, 624)])
    pltpu.sync_copy(zbuf.at[pl.ds(0, 624)], deg_out.at[pl.ds(obase, 624)])

    @pl.when(s == 0)
    def _():
        otail = pl.multiple_of(c * N + 9984, 8)
        pltpu.sync_copy(deg_sp.at[pl.ds(9984, 16)], zbuf.at[pl.ds(624, 16)])
        pltpu.sync_copy(zbuf.at[pl.ds(624, 16)], deg_out.at[pl.ds(otail, 16)])


@functools.partial(
    pl.kernel, mesh=_mesh,
    out_type=jax.ShapeDtypeStruct((2 * N, F), jnp.float32),
    scratch_types=[
        pltpu.VMEM((CH,), jnp.int32),
        pltpu.VMEM((CH,), jnp.int32),
        pltpu.VMEM((CH,), jnp.float32),
        pltpu.VMEM((K,), jnp.int32),
        pltpu.VMEM((K,), jnp.int32),
        pltpu.VMEM((K, F), jnp.float32),
        pltpu.VMEM((K, F), jnp.float32),
        pltpu.VMEM_SHARED((N, F), jnp.float32),
        pltpu.SemaphoreType.DMA,
        pltpu.SemaphoreType.DMA,
        pltpu.SemaphoreType.DMA,
        pltpu.SemaphoreType.DMA,
    ],
)
def _edge_agg(src_hbm, dst_hbm, w_hbm, h_hbm, agg_out,
              sidx_all, didx_all, wv_all, didx0, didx1,
              rows0, rows1, acc_sp, semG0, semG1, semS0, semS1):
    c = lax.axis_index("c")
    s = lax.axis_index("s")
    wid = s * 2 + c
    rowb = (rows0, rows1)
    didxb = (didx0, didx1)
    semG = (semG0, semG1)
    semS = (semS0, semS1)

    chunk = pl.multiple_of(wid * CH, 8)
    pltpu.sync_copy(src_hbm.at[pl.ds(chunk, CH)], sidx_all)
    pltpu.sync_copy(dst_hbm.at[pl.ds(chunk, CH)], didx_all)
    pltpu.sync_copy(w_hbm.at[pl.ds(chunk, CH)], wv_all)
    pltpu.async_copy(h_hbm.at[sidx_all.at[pl.ds(0, K)]], rows0, semG0)

    zero = jnp.zeros((16,), jnp.float32)

    def zbody(r, carry):
        for q in range(F // 16):
            rows1[r, pl.ds(q * 16, 16)] = zero
        return carry

    lax.fori_loop(0, K, zbody, 0)
    rstart = pl.multiple_of(s * 624, 8)
    for jj in range(8):
        size = 80 if jj < 7 else 64
        pltpu.sync_copy(rows1.at[pl.ds(0, size)],
                        acc_sp.at[pl.ds(rstart + jj * 80, size)])

    @pl.when(s == 15)
    def _():
        pltpu.sync_copy(rows1.at[pl.ds(0, 16)], acc_sp.at[pl.ds(9984, 16)])

    plsc.subcore_barrier()

    def slot(t, rb):
        rows = rowb[rb]
        off = pl.multiple_of(t * K, 16)
        pltpu.make_async_copy(h_hbm.at[sidx_all.at[pl.ds(0, K)]],
                              rows, semG[rb]).wait()
        for g in range(K // 16):
            didxb[rb][pl.ds(g * 16, 16)] = didx_all[pl.ds(off + g * 16, 16)]

        def mbody(g, mc):
            wg = wv_all[pl.ds(off + g * 16, 16)]
            for l in range(16):
                wspl = jnp.broadcast_to(wg[l], (16,))
                r = g * 16 + l
                for q in range(F // 16):
                    sl = pl.ds(q * 16, 16)
                    rows[r, sl] = rows[r, sl] * wspl
            return mc

        lax.fori_loop(0, K // 16, mbody, 0)

        @pl.when(t >= 1)
        def _():
            pltpu.make_async_copy(rowb[1 - rb], acc_sp.at[didxb[1 - rb]],
                                  semS[1 - rb]).wait()

        @pl.when(t + 1 <= WPW - 1)
        def _():
            off1 = pl.multiple_of((t + 1) * K, 16)
            pltpu.async_copy(h_hbm.at[sidx_all.at[pl.ds(off1, K)]],
                             rowb[1 - rb], semG[1 - rb])

        pltpu.async_copy(rows, acc_sp.at[didxb[rb]], semS[rb], add=True)

    def body(i, carry):
        t = i * 2
        slot(t, 0)
        slot(t + 1, 1)
        return carry

    lax.fori_loop(0, (WPW - 1) // 2, body, 0)
    slot(WPW - 1, 0)
    pltpu.make_async_copy(rowb[0], acc_sp.at[didxb[0]], semS[0]).wait()
    plsc.subcore_barrier()

    for jj in range(8):
        size = 80 if jj < 7 else 64
        ro = rstart + jj * 80
        oo = pl.multiple_of(c * N + ro, 8)
        pltpu.sync_copy(acc_sp.at[pl.ds(ro, size)], rows1.at[pl.ds(0, size)])
        pltpu.sync_copy(rows1.at[pl.ds(0, size)], agg_out.at[pl.ds(oo, size)])

    @pl.when(s == 15)
    def _():
        oo = pl.multiple_of(c * N + 9984, 8)
        pltpu.sync_copy(acc_sp.at[pl.ds(9984, 16)], rows1.at[pl.ds(0, 16)])
        pltpu.sync_copy(rows1.at[pl.ds(0, 16)], agg_out.at[pl.ds(oo, 16)])


def _tc_a_body(x_ref, wcat_ref, b_ref, w1_ref, a12_ref, h1pre_ref):
    x = x_ref[...]
    a12 = lax.dot_general(wcat_ref[...], x, (((1,), (1,)), ((), ())),
                          preferred_element_type=jnp.float32)
    a12_ref[...] = a12 + jnp.concatenate(
        [jnp.zeros((1, 1), jnp.float32), b_ref[...]], axis=0)
    h1pre_ref[...] = jnp.dot(x, w1_ref[...], preferred_element_type=jnp.float32)


def _tc_c_body(degp_ref, h1pre_ref, dinv_ref, h1s_ref):
    deg = degp_ref[:, 0:1] + degp_ref[:, 1:2] + 1.0
    dinv = lax.rsqrt(deg)
    dinv_ref[...] = dinv
    h1s_ref[...] = h1pre_ref[...] * dinv


def _bn_relu(conv, g_ref, beta_ref):
    r = jnp.maximum(conv, 0.0)
    mu = jnp.mean(r, axis=0, keepdims=True)
    var = jnp.mean(r * r, axis=0, keepdims=True) - mu * mu
    return g_ref[...] * (r - mu) * lax.rsqrt(var + 1e-5) + beta_ref[...]


def _tc_e_body(aggp_ref, h1pre_ref, dinv_ref, b1_ref, g1_ref, beta1_ref,
               w2_ref, h_ref, h2pre_ref, h2s_ref):
    dinv = dinv_ref[...]
    conv1 = dinv * (aggp_ref[0:N] + aggp_ref[N:2 * N]) \
        + (dinv * dinv) * h1pre_ref[...] + b1_ref[...]
    h = _bn_relu(conv1, g1_ref, beta1_ref)
    h_ref[...] = h
    h2pre = jnp.dot(h, w2_ref[...], preferred_element_type=jnp.float32)
    h2pre_ref[...] = h2pre
    h2s_ref[...] = h2pre * dinv


def _tc_g_body(aggp_ref, h2pre_ref, dinv_ref, b2_ref, g2_ref, beta2_ref,
               x_ref, h_ref, wf1_ref, bf1_ref, wf2_ref, bf2_ref, out_ref):
    dinv = dinv_ref[...]
    conv2 = dinv * (aggp_ref[0:N] + aggp_ref[N:2 * N]) \
        + (dinv * dinv) * h2pre_ref[...] + b2_ref[...]
    h2 = _bn_relu(conv2, g2_ref, beta2_ref)
    wf1 = wf1_ref[...]
    z = jnp.dot(x_ref[...], wf1[0:F], preferred_element_type=jnp.float32) \
        + jnp.dot(h_ref[...], wf1[F:2 * F], preferred_element_type=jnp.float32) \
        + jnp.dot(h2, wf1[2 * F:3 * F], preferred_element_type=jnp.float32) \
        + bf1_ref[...]
    z = jnp.maximum(z, 0.0)
    out = jnp.dot(z, wf2_ref[...], preferred_element_type=jnp.float32) + bf2_ref[...]
    out_ref[...] = jnp.maximum(out, 0.0)


_tc_a = pl.pallas_call(
    _tc_a_body,
    out_shape=(jax.ShapeDtypeStruct((2, N), jnp.float32),
               jax.ShapeDtypeStruct((N, F), jnp.float32)))

_tc_c = pl.pallas_call(
    _tc_c_body,
    out_shape=(jax.ShapeDtypeStruct((N, 1), jnp.float32),
               jax.ShapeDtypeStruct((N, F), jnp.float32)))

_tc_e = pl.pallas_call(
    _tc_e_body,
    out_shape=(jax.ShapeDtypeStruct((N, F), jnp.float32),
               jax.ShapeDtypeStruct((N, F), jnp.float32),
               jax.ShapeDtypeStruct((N, F), jnp.float32)))

_tc_g = pl.pallas_call(
    _tc_g_body,
    out_shape=jax.ShapeDtypeStruct((N, F), jnp.float32))


def kernel(x, edge_index, edge_weight, W_attn, b_attn, W1, b1, W2, b2,
           g1, beta1, g2, beta2, Wf1, bf1, Wf2, bf2):
    src3 = edge_index[0].astype(jnp.int32)
    dst3 = edge_index[1].astype(jnp.int32)
    wcat = jnp.stack([W_attn[:F, 0], W_attn[F:, 0]])
    a12, h1pre = _tc_a(x, wcat, b_attn.reshape(1, 1), W1)
    a1 = a12[0]
    a2 = a12[1]
    w_e, degp = _edge_w_deg(src3, dst3, edge_weight, a1, a2)
    dinv_col, h1s = _tc_c(degp.reshape(2, N).T, h1pre)
    agg1 = _edge_agg(src3, dst3, w_e, h1s)
    h, h2pre, h2s = _tc_e(agg1, h1pre, dinv_col, b1.reshape(1, F),
                          g1.reshape(1, F), beta1.reshape(1, F), W2)
    agg2 = _edge_agg(src3, dst3, w_e, h2s)
    out = _tc_g(agg2, h2pre, dinv_col, b2.reshape(1, F), g2.reshape(1, F),
                beta2.reshape(1, F), x, h, Wf1, bf1.reshape(1, F),
                Wf2, bf2.reshape(1, F))
    return out


# speedup vs baseline: 1.5787x; 1.5787x over previous
import functools

import jax
import jax.numpy as jnp
from jax import lax
from jax.experimental import pallas as pl
from jax.experimental.pallas import tpu as pltpu
from jax.experimental.pallas import tpu_sc as plsc

N = 10000
E = 320000
F = 128
NW = 32
CH = E // NW
K = 80
WPW = CH // K
KB = 400
WPWB = CH // KB

_mesh = plsc.VectorSubcoreMesh(core_axis_name="c", subcore_axis_name="s")


@functools.partial(
    pl.kernel, mesh=_mesh,
    out_type=(jax.ShapeDtypeStruct((E,), jnp.float32),
              jax.ShapeDtypeStruct((2 * N,), jnp.float32)),
    scratch_types=[
        pltpu.VMEM((CH,), jnp.int32),
        pltpu.VMEM((CH,), jnp.int32),
        pltpu.VMEM((CH,), jnp.float32),
        pltpu.VMEM((CH,), jnp.float32),
        pltpu.VMEM((KB,), jnp.int32),
        pltpu.VMEM((KB,), jnp.int32),
        pltpu.VMEM((KB,), jnp.float32),
        pltpu.VMEM((KB,), jnp.float32),
        pltpu.VMEM((KB,), jnp.float32),
        pltpu.VMEM((KB,), jnp.float32),
        pltpu.VMEM((640,), jnp.float32),
        pltpu.VMEM_SHARED((N,), jnp.float32),
        pltpu.SemaphoreType.DMA,
        pltpu.SemaphoreType.DMA,
        pltpu.SemaphoreType.DMA,
        pltpu.SemaphoreType.DMA,
        pltpu.SemaphoreType.DMA,
        pltpu.SemaphoreType.DMA,
    ],
)
def _edge_w_deg(src_hbm, dst_hbm, ew_hbm, a1_hbm, a2_hbm, w_out, deg_out,
                sidx_all, didx_all, ewv, wv_all, didx0, didx1,
                a1v0, a1v1, a2v0, a2v1,
                zbuf, deg_sp, semA0, semA1, semB0, semB1, semD0, semD1):
    c = lax.axis_index("c")
    s = lax.axis_index("s")
    wid = s * 2 + c
    a1b = (a1v0, a1v1)
    a2b = (a2v0, a2v1)
    didxb = (didx0, didx1)
    semA = (semA0, semA1)
    semB = (semB0, semB1)
    semD = (semD0, semD1)

    chunk = pl.multiple_of(wid * CH, 8)
    pltpu.sync_copy(src_hbm.at[pl.ds(chunk, CH)], sidx_all)
    pltpu.sync_copy(dst_hbm.at[pl.ds(chunk, CH)], didx_all)
    pltpu.sync_copy(ew_hbm.at[pl.ds(chunk, CH)], ewv)
    pltpu.async_copy(a1_hbm.at[sidx_all.at[pl.ds(0, KB)]], a1v0, semA0)
    pltpu.async_copy(a2_hbm.at[didx_all.at[pl.ds(0, KB)]], a2v0, semB0)

    zero = jnp.zeros((16,), jnp.float32)
    for q in range(40):
        zbuf[pl.ds(q * 16, 16)] = zero
    base = pl.multiple_of(s * 624, 8)
    pltpu.sync_copy(zbuf.at[pl.ds(0, 624)], deg_sp.at[pl.ds(base, 624)])

    @pl.when(s == 0)
    def _():
        pltpu.sync_copy(zbuf.at[pl.ds(0, 16)], deg_sp.at[pl.ds(9984, 16)])

    plsc.subcore_barrier()

    def slot(t, rb, prefetch=True, warmup=False):
        off = pl.multiple_of(t * KB, 16)
        pltpu.make_async_copy(a1_hbm.at[sidx_all.at[pl.ds(0, KB)]],
                              a1b[rb], semA[rb]).wait()
        pltpu.make_async_copy(a2_hbm.at[didx_all.at[pl.ds(0, KB)]],
                              a2b[rb], semB[rb]).wait()
        if prefetch:
            off1 = pl.multiple_of((t + 1) * KB, 16)
            pltpu.async_copy(a1_hbm.at[sidx_all.at[pl.ds(off1, KB)]],
                             a1b[1 - rb], semA[1 - rb])
            pltpu.async_copy(a2_hbm.at[didx_all.at[pl.ds(off1, KB)]],
                             a2b[1 - rb], semB[1 - rb])
        for g in range(KB // 16):
            sl = pl.ds(g * 16, 16)
            z = a1b[rb][sl] + a2b[rb][sl]
            wv_all[pl.ds(off + g * 16, 16)] = ewv[pl.ds(off + g * 16, 16)] \
                / (1.0 + jnp.exp(-z))
        if not warmup:
            pltpu.make_async_copy(wv_all.at[pl.ds(0, KB)],
                                  deg_sp.at[didxb[rb]], semD[rb]).wait()
        for g in range(KB // 16):
            didxb[rb][pl.ds(g * 16, 16)] = didx_all[pl.ds(off + g * 16, 16)]
        pltpu.async_copy(wv_all.at[pl.ds(off, KB)], deg_sp.at[didxb[rb]],
                         semD[rb], add=True)

    slot(0, 0, warmup=True)
    slot(1, 1, warmup=True)

    def body(i, carry):
        t = i * 2 + 2
        slot(t, 0)
        slot(t + 1, 1)
        return carry

    lax.fori_loop(0, (WPWB - 3) // 2, body, 0)
    slot(WPWB - 1, 0, prefetch=False)
    pltpu.make_async_copy(wv_all.at[pl.ds(0, KB)], deg_sp.at[didxb[1]],
                          semD[1]).wait()
    pltpu.make_async_copy(wv_all.at[pl.ds(0, KB)], deg_sp.at[didxb[0]],
                          semD[0]).wait()

    pltpu.sync_copy(wv_all, w_out.at[pl.ds(chunk, CH)])
    plsc.subcore_barrier()

    obase = pl.multiple_of(c * N + s * 624, 8)
    pltpu.sync_copy(deg_sp.at[pl.ds(base, 624)], zbuf.at[pl.ds(0, 624)])
    pltpu.sync_copy(zbuf.at[pl.ds(0, 624)], deg_out.at[pl.ds(obase, 624)])

    @pl.when(s == 0)
    def _():
        otail = pl.multiple_of(c * N + 9984, 8)
        pltpu.sync_copy(deg_sp.at[pl.ds(9984, 16)], zbuf.at[pl.ds(624, 16)])
        pltpu.sync_copy(zbuf.at[pl.ds(624, 16)], deg_out.at[pl.ds(otail, 16)])


@functools.partial(
    pl.kernel, mesh=_mesh,
    out_type=jax.ShapeDtypeStruct((2 * N, F), jnp.float32),
    scratch_types=[
        pltpu.VMEM((CH,), jnp.int32),
        pltpu.VMEM((K,), jnp.float32),
        pltpu.VMEM((K,), jnp.float32),
        pltpu.VMEM((K,), jnp.float32),
        pltpu.VMEM((K,), jnp.int32),
        pltpu.VMEM((K,), jnp.int32),
        pltpu.VMEM((K,), jnp.int32),
        pltpu.VMEM((K, F), jnp.float32),
        pltpu.VMEM((K, F), jnp.float32),
        pltpu.VMEM((K, F), jnp.float32),
        pltpu.VMEM_SHARED((N, F), jnp.float32),
        pltpu.SemaphoreType.DMA,
        pltpu.SemaphoreType.DMA,
        pltpu.SemaphoreType.DMA,
        pltpu.SemaphoreType.DMA,
        pltpu.SemaphoreType.DMA,
        pltpu.SemaphoreType.DMA,
        pltpu.SemaphoreType.DMA,
        pltpu.SemaphoreType.DMA,
        pltpu.SemaphoreType.DMA,
        pltpu.SemaphoreType.DMA,
        pltpu.SemaphoreType.DMA,
        pltpu.SemaphoreType.DMA,
    ],
)
def _edge_agg(src_hbm, dst_hbm, w_hbm, h_hbm, agg_out,
              sidx_all, wv0, wv1, wv2, didx0, didx1, didx2,
              rows0, rows1, rows2, acc_sp,
              semG0, semG1, semG2, semS0, semS1, semS2,
              semI0, semI1, semI2, semW0, semW1, semW2):
    c = lax.axis_index("c")
    s = lax.axis_index("s")
    wid = s * 2 + c
    rowb = (rows0, rows1, rows2)
    wvb = (wv0, wv1, wv2)
    didxb = (didx0, didx1, didx2)
    semG = (semG0, semG1, semG2)
    semS = (semS0, semS1, semS2)
    semI = (semI0, semI1, semI2)
    semW = (semW0, semW1, semW2)

    chunk = pl.multiple_of(wid * CH, 8)
    pltpu.sync_copy(src_hbm.at[pl.ds(chunk, CH)], sidx_all)
    pltpu.async_copy(h_hbm.at[sidx_all.at[pl.ds(0, K)]], rows0, semG0)
    pltpu.async_copy(h_hbm.at[sidx_all.at[pl.ds(K, K)]], rows1, semG1)
    pltpu.async_copy(dst_hbm.at[pl.ds(chunk, K)], didx0, semI0)
    pltpu.async_copy(dst_hbm.at[pl.ds(pl.multiple_of(chunk + K, 8), K)],
                     didx1, semI1)
    pltpu.async_copy(w_hbm.at[pl.ds(chunk, K)], wv0, semW0)
    pltpu.async_copy(w_hbm.at[pl.ds(pl.multiple_of(chunk + K, 8), K)],
                     wv1, semW1)

    zero = jnp.zeros((16,), jnp.float32)

    def zbody(r, carry):
        for q in range(F // 16):
            rows2[r, pl.ds(q * 16, 16)] = zero
        return carry

    lax.fori_loop(0, K, zbody, 0)
    rstart = pl.multiple_of(s * 624, 8)
    for jj in range(8):
        size = 80 if jj < 7 else 64
        pltpu.sync_copy(rows2.at[pl.ds(0, size)],
                        acc_sp.at[pl.ds(rstart + jj * 80, size)])

    @pl.when(s == 15)
    def _():
        pltpu.sync_copy(rows2.at[pl.ds(0, 16)], acc_sp.at[pl.ds(9984, 16)])

    plsc.subcore_barrier()

    def slot(t, rb, wait_scatter=True, prefetch=True):
        nb = (rb + 2) % 3
        rows = rowb[rb]
        pltpu.make_async_copy(h_hbm.at[sidx_all.at[pl.ds(0, K)]],
                              rows, semG[rb]).wait()
        pltpu.make_async_copy(w_hbm.at[pl.ds(chunk, K)], wvb[rb],
                              semW[rb]).wait()

        def mbody(g, mc):
            wg = wvb[rb][pl.ds(g * 16, 16)]
            for l in range(16):
                wspl = jnp.broadcast_to(wg[l], (16,))
                r = g * 16 + l
                for q in range(F // 16):
                    sl = pl.ds(q * 16, 16)
                    rows[r, sl] = rows[r, sl] * wspl
            return mc

        lax.fori_loop(0, K // 16, mbody, 0)

        if wait_scatter:
            pltpu.make_async_copy(rowb[nb], acc_sp.at[didxb[nb]],
                                  semS[nb]).wait()

        def issue_gather():
            off2 = pl.multiple_of((t + 2) * K, 16)
            pltpu.async_copy(h_hbm.at[sidx_all.at[pl.ds(off2, K)]],
                             rowb[nb], semG[nb])
            pltpu.async_copy(
                dst_hbm.at[pl.ds(pl.multiple_of(chunk + off2, 8), K)],
                didxb[nb], semI[nb])
            pltpu.async_copy(
                w_hbm.at[pl.ds(pl.multiple_of(chunk + off2, 8), K)],
                wvb[nb], semW[nb])

        if prefetch:
            if isinstance(t, int):
                issue_gather()
            else:
                pl.when(t + 2 <= WPW - 1)(issue_gather)

        pltpu.make_async_copy(dst_hbm.at[pl.ds(chunk, K)], didxb[rb],
                              semI[rb]).wait()
        pltpu.async_copy(rows, acc_sp.at[didxb[rb]], semS[rb], add=True)

    slot(0, 0, wait_scatter=False)
    slot(1, 1)
    slot(2, 2)

    def body(i, carry):
        t = i * 3 + 3
        slot(t, 0)
        slot(t + 1, 1)
        slot(t + 2, 2)
        return carry

    lax.fori_loop(0, (WPW - 5) // 3, body, 0)
    slot(WPW - 2, 0, prefetch=False)
    slot(WPW - 1, 1, prefetch=False)
    pltpu.make_async_copy(rowb[1], acc_sp.at[didxb[1]], semS[1]).wait()
    plsc.subcore_barrier()

    for jj in range(8):
        size = 80 if jj < 7 else 64
        ro = rstart + jj * 80
        oo = pl.multiple_of(c * N + ro, 8)
        pltpu.sync_copy(acc_sp.at[pl.ds(ro, size)], rows1.at[pl.ds(0, size)])
        pltpu.sync_copy(rows1.at[pl.ds(0, size)], agg_out.at[pl.ds(oo, size)])

    @pl.when(s == 15)
    def _():
        oo = pl.multiple_of(c * N + 9984, 8)
        pltpu.sync_copy(acc_sp.at[pl.ds(9984, 16)], rows1.at[pl.ds(0, 16)])
        pltpu.sync_copy(rows1.at[pl.ds(0, 16)], agg_out.at[pl.ds(oo, 16)])


def _tc_a_body(x_ref, wcat_ref, b_ref, w1_ref, a12_ref, h1pre_ref):
    x = x_ref[...]
    a12 = lax.dot_general(wcat_ref[...], x, (((1,), (1,)), ((), ())),
                          preferred_element_type=jnp.float32)
    a12_ref[...] = a12 + jnp.concatenate(
        [jnp.zeros((1, 1), jnp.float32), b_ref[...]], axis=0)
    h1pre_ref[...] = jnp.dot(x, w1_ref[...], preferred_element_type=jnp.float32)


def _tc_c_body(degp_ref, h1pre_ref, dinv_ref, h1s_ref):
    deg = degp_ref[:, 0:1] + degp_ref[:, 1:2] + 1.0
    dinv = lax.rsqrt(deg)
    dinv_ref[...] = dinv
    h1s_ref[...] = h1pre_ref[...] * dinv


def _bn_relu(conv, g_ref, beta_ref):
    r = jnp.maximum(conv, 0.0)
    mu = jnp.mean(r, axis=0, keepdims=True)
    var = jnp.mean(r * r, axis=0, keepdims=True) - mu * mu
    return g_ref[...] * (r - mu) * lax.rsqrt(var + 1e-5) + beta_ref[...]


def _tc_e_body(aggp_ref, h1pre_ref, dinv_ref, b1_ref, g1_ref, beta1_ref,
               w2_ref, h_ref, h2pre_ref, h2s_ref):
    dinv = dinv_ref[...]
    conv1 = dinv * (aggp_ref[0:N] + aggp_ref[N:2 * N]) \
        + (dinv * dinv) * h1pre_ref[...] + b1_ref[...]
    h = _bn_relu(conv1, g1_ref, beta1_ref)
    h_ref[...] = h
    h2pre = jnp.dot(h, w2_ref[...], preferred_element_type=jnp.float32)
    h2pre_ref[...] = h2pre
    h2s_ref[...] = h2pre * dinv


def _tc_g_body(aggp_ref, h2pre_ref, dinv_ref, b2_ref, g2_ref, beta2_ref,
               x_ref, h_ref, wf1_ref, bf1_ref, wf2_ref, bf2_ref, out_ref):
    dinv = dinv_ref[...]
    conv2 = dinv * (aggp_ref[0:N] + aggp_ref[N:2 * N]) \
        + (dinv * dinv) * h2pre_ref[...] + b2_ref[...]
    h2 = _bn_relu(conv2, g2_ref, beta2_ref)
    wf1 = wf1_ref[...]
    z = jnp.dot(x_ref[...], wf1[0:F], preferred_element_type=jnp.float32) \
        + jnp.dot(h_ref[...], wf1[F:2 * F], preferred_element_type=jnp.float32) \
        + jnp.dot(h2, wf1[2 * F:3 * F], preferred_element_type=jnp.float32) \
        + bf1_ref[...]
    z = jnp.maximum(z, 0.0)
    out = jnp.dot(z, wf2_ref[...], preferred_element_type=jnp.float32) + bf2_ref[...]
    out_ref[...] = jnp.maximum(out, 0.0)


_tc_a = pl.pallas_call(
    _tc_a_body,
    out_shape=(jax.ShapeDtypeStruct((2, N), jnp.float32),
               jax.ShapeDtypeStruct((N, F), jnp.float32)))

_tc_c = pl.pallas_call(
    _tc_c_body,
    out_shape=(jax.ShapeDtypeStruct((N, 1), jnp.float32),
               jax.ShapeDtypeStruct((N, F), jnp.float32)))

_tc_e = pl.pallas_call(
    _tc_e_body,
    out_shape=(jax.ShapeDtypeStruct((N, F), jnp.float32),
               jax.ShapeDtypeStruct((N, F), jnp.float32),
               jax.ShapeDtypeStruct((N, F), jnp.float32)))

_tc_g = pl.pallas_call(
    _tc_g_body,
    out_shape=jax.ShapeDtypeStruct((N, F), jnp.float32))


def kernel(x, edge_index, edge_weight, W_attn, b_attn, W1, b1, W2, b2,
           g1, beta1, g2, beta2, Wf1, bf1, Wf2, bf2):
    src3 = edge_index[0].astype(jnp.int32)
    dst3 = edge_index[1].astype(jnp.int32)
    wcat = jnp.stack([W_attn[:F, 0], W_attn[F:, 0]])
    a12, h1pre = _tc_a(x, wcat, b_attn.reshape(1, 1), W1)
    a1 = a12[0]
    a2 = a12[1]
    w_e, degp = _edge_w_deg(src3, dst3, edge_weight, a1, a2)
    dinv_col, h1s = _tc_c(degp.reshape(2, N).T, h1pre)
    agg1 = _edge_agg(src3, dst3, w_e, h1s)
    h, h2pre, h2s = _tc_e(agg1, h1pre, dinv_col, b1.reshape(1, F),
                          g1.reshape(1, F), beta1.reshape(1, F), W2)
    agg2 = _edge_agg(src3, dst3, w_e, h2s)
    out = _tc_g(agg2, h2pre, dinv_col, b2.reshape(1, F), g2.reshape(1, F),
                beta2.reshape(1, F), x, h, Wf1, bf1.reshape(1, F),
                Wf2, bf2.reshape(1, F))
    return out


# speedup vs baseline: 1.6397x; 1.0386x over previous
import functools

import jax
import jax.numpy as jnp
from jax import lax
from jax.experimental import pallas as pl
from jax.experimental.pallas import tpu as pltpu
from jax.experimental.pallas import tpu_sc as plsc

N = 10000
E = 320000
F = 128
NW = 32
CH = E // NW
K = 80
WPW = CH // K
KB = 400
WPWB = CH // KB

_mesh = plsc.VectorSubcoreMesh(core_axis_name="c", subcore_axis_name="s")


@functools.partial(
    pl.kernel, mesh=_mesh,
    out_type=(jax.ShapeDtypeStruct((E,), jnp.float32),
              jax.ShapeDtypeStruct((2 * N,), jnp.float32)),
    scratch_types=[
        pltpu.VMEM((CH,), jnp.int32),
        pltpu.VMEM((CH,), jnp.int32),
        pltpu.VMEM((CH,), jnp.float32),
        pltpu.VMEM((CH,), jnp.float32),
        pltpu.VMEM((KB,), jnp.int32),
        pltpu.VMEM((KB,), jnp.int32),
        pltpu.VMEM((KB,), jnp.float32),
        pltpu.VMEM((KB,), jnp.float32),
        pltpu.VMEM((KB,), jnp.float32),
        pltpu.VMEM((KB,), jnp.float32),
        pltpu.VMEM((640,), jnp.float32),
        pltpu.VMEM_SHARED((N,), jnp.float32),
        pltpu.SemaphoreType.DMA,
        pltpu.SemaphoreType.DMA,
        pltpu.SemaphoreType.DMA,
        pltpu.SemaphoreType.DMA,
        pltpu.SemaphoreType.DMA,
        pltpu.SemaphoreType.DMA,
    ],
)
def _edge_w_deg(src_hbm, dst_hbm, ew_hbm, a1_hbm, a2_hbm, w_out, deg_out,
                sidx_all, didx_all, ewv, wv_all, didx0, didx1,
                a1v0, a1v1, a2v0, a2v1,
                zbuf, deg_sp, semA0, semA1, semB0, semB1, semD0, semD1):
    c = lax.axis_index("c")
    s = lax.axis_index("s")
    wid = s * 2 + c
    a1b = (a1v0, a1v1)
    a2b = (a2v0, a2v1)
    didxb = (didx0, didx1)
    semA = (semA0, semA1)
    semB = (semB0, semB1)
    semD = (semD0, semD1)

    chunk = pl.multiple_of(wid * CH, 8)
    pltpu.sync_copy(src_hbm.at[pl.ds(chunk, CH)], sidx_all)
    pltpu.sync_copy(dst_hbm.at[pl.ds(chunk, CH)], didx_all)
    pltpu.sync_copy(ew_hbm.at[pl.ds(chunk, CH)], ewv)
    pltpu.async_copy(a1_hbm.at[sidx_all.at[pl.ds(0, KB)]], a1v0, semA0)
    pltpu.async_copy(a2_hbm.at[didx_all.at[pl.ds(0, KB)]], a2v0, semB0)

    zero = jnp.zeros((16,), jnp.float32)
    for q in range(40):
        zbuf[pl.ds(q * 16, 16)] = zero
    base = pl.multiple_of(s * 624, 8)
    pltpu.sync_copy(zbuf.at[pl.ds(0, 624)], deg_sp.at[pl.ds(base, 624)])

    @pl.when(s == 0)
    def _():
        pltpu.sync_copy(zbuf.at[pl.ds(0, 16)], deg_sp.at[pl.ds(9984, 16)])

    plsc.subcore_barrier()

    def slot(t, rb, prefetch=True, warmup=False):
        off = pl.multiple_of(t * KB, 16)
        pltpu.make_async_copy(a1_hbm.at[sidx_all.at[pl.ds(0, KB)]],
                              a1b[rb], semA[rb]).wait()
        pltpu.make_async_copy(a2_hbm.at[didx_all.at[pl.ds(0, KB)]],
                              a2b[rb], semB[rb]).wait()
        if prefetch:
            off1 = pl.multiple_of((t + 1) * KB, 16)
            pltpu.async_copy(a1_hbm.at[sidx_all.at[pl.ds(off1, KB)]],
                             a1b[1 - rb], semA[1 - rb])
            pltpu.async_copy(a2_hbm.at[didx_all.at[pl.ds(off1, KB)]],
                             a2b[1 - rb], semB[1 - rb])
        for g in range(KB // 16):
            sl = pl.ds(g * 16, 16)
            z = a1b[rb][sl] + a2b[rb][sl]
            wv_all[pl.ds(off + g * 16, 16)] = ewv[pl.ds(off + g * 16, 16)] \
                / (1.0 + jnp.exp(-z))
        if not warmup:
            pltpu.make_async_copy(wv_all.at[pl.ds(0, KB)],
                                  deg_sp.at[didxb[rb]], semD[rb]).wait()
        for g in range(KB // 16):
            didxb[rb][pl.ds(g * 16, 16)] = didx_all[pl.ds(off + g * 16, 16)]
        pltpu.async_copy(wv_all.at[pl.ds(off, KB)], deg_sp.at[didxb[rb]],
                         semD[rb], add=True)

    slot(0, 0, warmup=True)
    slot(1, 1, warmup=True)

    def body(i, carry):
        t = i * 2 + 2
        slot(t, 0)
        slot(t + 1, 1)
        return carry

    lax.fori_loop(0, (WPWB - 3) // 2, body, 0)
    slot(WPWB - 1, 0, prefetch=False)
    pltpu.make_async_copy(wv_all.at[pl.ds(0, KB)], deg_sp.at[didxb[1]],
                          semD[1]).wait()
    pltpu.make_async_copy(wv_all.at[pl.ds(0, KB)], deg_sp.at[didxb[0]],
                          semD[0]).wait()

    pltpu.sync_copy(wv_all, w_out.at[pl.ds(chunk, CH)])
    plsc.subcore_barrier()

    obase = pl.multiple_of(c * N + s * 624, 8)
    pltpu.sync_copy(deg_sp.at[pl.ds(base, 624)], zbuf.at[pl.ds(0, 624)])
    pltpu.sync_copy(zbuf.at[pl.ds(0, 624)], deg_out.at[pl.ds(obase, 624)])

    @pl.when(s == 0)
    def _():
        otail = pl.multiple_of(c * N + 9984, 8)
        pltpu.sync_copy(deg_sp.at[pl.ds(9984, 16)], zbuf.at[pl.ds(624, 16)])
        pltpu.sync_copy(zbuf.at[pl.ds(624, 16)], deg_out.at[pl.ds(otail, 16)])


@functools.partial(
    pl.kernel, mesh=_mesh,
    out_type=jax.ShapeDtypeStruct((2 * N, F), jnp.float32),
    scratch_types=(
        [pltpu.VMEM((K,), jnp.int32) for _ in range(4)] +
        [pltpu.VMEM((K,), jnp.float32) for _ in range(4)] +
        [pltpu.VMEM((K,), jnp.int32) for _ in range(4)] +
        [pltpu.VMEM((K, F), jnp.float32) for _ in range(4)] +
        [pltpu.VMEM_SHARED((N, F), jnp.float32)] +
        [pltpu.SemaphoreType.DMA for _ in range(20)]
    ),
)
def _edge_agg(src_hbm, dst_hbm, w_hbm, h_hbm, agg_out,
              sx0, sx1, sx2, sx3, wv0, wv1, wv2, wv3,
              didx0, didx1, didx2, didx3,
              rows0, rows1, rows2, rows3, acc_sp, *sems):
    c = lax.axis_index("c")
    s = lax.axis_index("s")
    wid = s * 2 + c
    sxb = (sx0, sx1, sx2, sx3)
    wvb = (wv0, wv1, wv2, wv3)
    didxb = (didx0, didx1, didx2, didx3)
    rowb = (rows0, rows1, rows2, rows3)
    semG = sems[0:4]
    semS = sems[4:8]
    semI = sems[8:12]
    semW = sems[12:16]
    semX = sems[16:20]

    chunk = pl.multiple_of(wid * CH, 8)

    def woff(t):
        return pl.ds(pl.multiple_of(chunk + t * K, 8), K)

    for w in range(3):
        pltpu.sync_copy(src_hbm.at[woff(w)], sxb[w])
        pltpu.async_copy(h_hbm.at[sxb[w].at[pl.ds(0, K)]], rowb[w], semG[w])
        pltpu.async_copy(dst_hbm.at[woff(w)], didxb[w], semI[w])
        pltpu.async_copy(w_hbm.at[woff(w)], wvb[w], semW[w])
    pltpu.async_copy(src_hbm.at[woff(3)], sx3, semX[3])

    zero = jnp.zeros((16,), jnp.float32)

    def zbody(r, carry):
        for q in range(F // 16):
            rows3[r, pl.ds(q * 16, 16)] = zero
        return carry

    lax.fori_loop(0, K, zbody, 0)
    rstart = pl.multiple_of(s * 624, 8)
    for jj in range(8):
        size = 80 if jj < 7 else 64
        pltpu.sync_copy(rows3.at[pl.ds(0, size)],
                        acc_sp.at[pl.ds(rstart + jj * 80, size)])

    @pl.when(s == 15)
    def _():
        pltpu.sync_copy(rows3.at[pl.ds(0, 16)], acc_sp.at[pl.ds(9984, 16)])

    plsc.subcore_barrier()

    def slot(t, rb, wait_scatter=True, prefetch=True):
        nb = (rb + 3) % 4
        rows = rowb[rb]
        pltpu.make_async_copy(h_hbm.at[sxb[rb].at[pl.ds(0, K)]],
                              rows, semG[rb]).wait()
        pltpu.make_async_copy(w_hbm.at[pl.ds(chunk, K)], wvb[rb],
                              semW[rb]).wait()

        def mbody(g, mc):
            wg = wvb[rb][pl.ds(g * 16, 16)]
            for l in range(16):
                wspl = jnp.broadcast_to(wg[l], (16,))
                r = g * 16 + l
                for q in range(F // 16):
                    sl = pl.ds(q * 16, 16)
                    rows[r, sl] = rows[r, sl] * wspl
            return mc

        lax.fori_loop(0, K // 16, mbody, 0)

        if wait_scatter:
            pltpu.make_async_copy(rowb[nb], acc_sp.at[didxb[nb]],
                                  semS[nb]).wait()

        def issue_gather():
            pltpu.make_async_copy(src_hbm.at[pl.ds(chunk, K)], sxb[nb],
                                  semX[nb]).wait()
            pltpu.async_copy(h_hbm.at[sxb[nb].at[pl.ds(0, K)]],
                             rowb[nb], semG[nb])
            pltpu.async_copy(dst_hbm.at[woff(t + 3)], didxb[nb], semI[nb])
            pltpu.async_copy(w_hbm.at[woff(t + 3)], wvb[nb], semW[nb])

            def issue_sidx():
                pltpu.async_copy(src_hbm.at[woff(t + 4)], sxb[rb], semX[rb])

            if isinstance(t, int):
                issue_sidx()
            else:
                pl.when(t + 4 <= WPW - 1)(issue_sidx)

        if prefetch:
            if isinstance(t, int):
                issue_gather()
            else:
                pl.when(t + 3 <= WPW - 1)(issue_gather)

        pltpu.make_async_copy(dst_hbm.at[pl.ds(chunk, K)], didxb[rb],
                              semI[rb]).wait()
        pltpu.async_copy(rows, acc_sp.at[didxb[rb]], semS[rb], add=True)

    slot(0, 0, wait_scatter=False)
    slot(1, 1)
    slot(2, 2)

    def body(i, carry):
        t = i * 4 + 3
        slot(t, 3)
        slot(t + 1, 0)
        slot(t + 2, 1)
        slot(t + 3, 2)
        return carry

    lax.fori_loop(0, (WPW - 5) // 4, body, 0)
    slot(WPW - 2, 3, prefetch=False)
    slot(WPW - 1, 0, prefetch=False)
    pltpu.make_async_copy(rowb[0], acc_sp.at[didxb[0]], semS[0]).wait()
    plsc.subcore_barrier()

    for jj in range(8):
        size = 80 if jj < 7 else 64
        ro = rstart + jj * 80
        oo = pl.multiple_of(c * N + ro, 8)
        pltpu.sync_copy(acc_sp.at[pl.ds(ro, size)], rows1.at[pl.ds(0, size)])
        pltpu.sync_copy(rows1.at[pl.ds(0, size)], agg_out.at[pl.ds(oo, size)])

    @pl.when(s == 15)
    def _():
        oo = pl.multiple_of(c * N + 9984, 8)
        pltpu.sync_copy(acc_sp.at[pl.ds(9984, 16)], rows1.at[pl.ds(0, 16)])
        pltpu.sync_copy(rows1.at[pl.ds(0, 16)], agg_out.at[pl.ds(oo, 16)])


def _tc_a_body(x_ref, wcat_ref, b_ref, w1_ref, a12_ref, h1pre_ref):
    x = x_ref[...]
    a12 = lax.dot_general(wcat_ref[...], x, (((1,), (1,)), ((), ())),
                          preferred_element_type=jnp.float32)
    a12_ref[...] = a12 + jnp.concatenate(
        [jnp.zeros((1, 1), jnp.float32), b_ref[...]], axis=0)
    h1pre_ref[...] = jnp.dot(x, w1_ref[...], preferred_element_type=jnp.float32)


def _tc_c_body(degp_ref, h1pre_ref, dinv_ref, h1s_ref):
    deg = degp_ref[:, 0:1] + degp_ref[:, 1:2] + 1.0
    dinv = lax.rsqrt(deg)
    dinv_ref[...] = dinv
    h1s_ref[...] = h1pre_ref[...] * dinv


def _bn_relu(conv, g_ref, beta_ref):
    r = jnp.maximum(conv, 0.0)
    mu = jnp.mean(r, axis=0, keepdims=True)
    var = jnp.mean(r * r, axis=0, keepdims=True) - mu * mu
    return g_ref[...] * (r - mu) * lax.rsqrt(var + 1e-5) + beta_ref[...]


def _tc_e_body(aggp_ref, h1pre_ref, dinv_ref, b1_ref, g1_ref, beta1_ref,
               w2_ref, h_ref, h2pre_ref, h2s_ref):
    dinv = dinv_ref[...]
    conv1 = dinv * (aggp_ref[0:N] + aggp_ref[N:2 * N]) \
        + (dinv * dinv) * h1pre_ref[...] + b1_ref[...]
    h = _bn_relu(conv1, g1_ref, beta1_ref)
    h_ref[...] = h
    h2pre = jnp.dot(h, w2_ref[...], preferred_element_type=jnp.float32)
    h2pre_ref[...] = h2pre
    h2s_ref[...] = h2pre * dinv


def _tc_g_body(aggp_ref, h2pre_ref, dinv_ref, b2_ref, g2_ref, beta2_ref,
               x_ref, h_ref, wf1_ref, bf1_ref, wf2_ref, bf2_ref, out_ref):
    dinv = dinv_ref[...]
    conv2 = dinv * (aggp_ref[0:N] + aggp_ref[N:2 * N]) \
        + (dinv * dinv) * h2pre_ref[...] + b2_ref[...]
    h2 = _bn_relu(conv2, g2_ref, beta2_ref)
    wf1 = wf1_ref[...]
    z = jnp.dot(x_ref[...], wf1[0:F], preferred_element_type=jnp.float32) \
        + jnp.dot(h_ref[...], wf1[F:2 * F], preferred_element_type=jnp.float32) \
        + jnp.dot(h2, wf1[2 * F:3 * F], preferred_element_type=jnp.float32) \
        + bf1_ref[...]
    z = jnp.maximum(z, 0.0)
    out = jnp.dot(z, wf2_ref[...], preferred_element_type=jnp.float32) + bf2_ref[...]
    out_ref[...] = jnp.maximum(out, 0.0)


_tc_a = pl.pallas_call(
    _tc_a_body,
    out_shape=(jax.ShapeDtypeStruct((2, N), jnp.float32),
               jax.ShapeDtypeStruct((N, F), jnp.float32)))

_tc_c = pl.pallas_call(
    _tc_c_body,
    out_shape=(jax.ShapeDtypeStruct((N, 1), jnp.float32),
               jax.ShapeDtypeStruct((N, F), jnp.float32)))

_tc_e = pl.pallas_call(
    _tc_e_body,
    out_shape=(jax.ShapeDtypeStruct((N, F), jnp.float32),
               jax.ShapeDtypeStruct((N, F), jnp.float32),
               jax.ShapeDtypeStruct((N, F), jnp.float32)))

_tc_g = pl.pallas_call(
    _tc_g_body,
    out_shape=jax.ShapeDtypeStruct((N, F), jnp.float32))


def kernel(x, edge_index, edge_weight, W_attn, b_attn, W1, b1, W2, b2,
           g1, beta1, g2, beta2, Wf1, bf1, Wf2, bf2):
    src3 = edge_index[0].astype(jnp.int32)
    dst3 = edge_index[1].astype(jnp.int32)
    wcat = jnp.stack([W_attn[:F, 0], W_attn[F:, 0]])
    a12, h1pre = _tc_a(x, wcat, b_attn.reshape(1, 1), W1)
    a1 = a12[0]
    a2 = a12[1]
    w_e, degp = _edge_w_deg(src3, dst3, edge_weight, a1, a2)
    dinv_col, h1s = _tc_c(degp.reshape(2, N).T, h1pre)
    agg1 = _edge_agg(src3, dst3, w_e, h1s)
    h, h2pre, h2s = _tc_e(agg1, h1pre, dinv_col, b1.reshape(1, F),
                          g1.reshape(1, F), beta1.reshape(1, F), W2)
    agg2 = _edge_agg(src3, dst3, w_e, h2s)
    out = _tc_g(agg2, h2pre, dinv_col, b2.reshape(1, F), g2.reshape(1, F),
                beta2.reshape(1, F), x, h, Wf1, bf1.reshape(1, F),
                Wf2, bf2.reshape(1, F))
    return out
